# Initial kernel scaffold; baseline (speedup 1.0000x reference)
import jax
import jax.numpy as jnp
from jax.experimental import pallas as pl


def kernel(input_tensor, label_tensor):
    raise NotImplementedError("write your pallas kernel here")


# speedup vs baseline: 174.9076x; 174.9076x over previous
import functools

import jax
import jax.numpy as jnp
from jax import lax
from jax.experimental import pallas as pl
from jax.experimental.pallas import tpu as pltpu
from jax.experimental.pallas import tpu_sc as plsc

NUM_LABELS = 64
N = 4 * 512 * 512
NC, NS, L = 2, 16, 16
NW = NC * NS
CH = N // NW


@functools.partial(
    pl.kernel,
    mesh=plsc.VectorSubcoreMesh(core_axis_name="c", subcore_axis_name="s"),
    out_type=jax.ShapeDtypeStruct((N,), jnp.float32),
    scratch_types=[
        pltpu.VMEM((CH,), jnp.float32),
        pltpu.VMEM((CH,), jnp.int32),
        pltpu.VMEM((2 * L,), jnp.int32),
    ],
)
def _clear_body(inp_hbm, lab_hbm, words_hbm, out_hbm, inp_v, lab_v, words_v):
    wid = lax.axis_index("s") * NC + lax.axis_index("c")
    base = wid * CH
    pltpu.sync_copy(words_hbm, words_v)
    pltpu.sync_copy(inp_hbm.at[pl.ds(base, CH)], inp_v)
    pltpu.sync_copy(lab_hbm.at[pl.ds(base, CH)], lab_v)
    lo = words_v[pl.ds(0, L)]
    hi = words_v[pl.ds(L, L)]

    def step(i, carry):
        off = i * L
        labs = lab_v[pl.ds(off, L)]
        word = jnp.where(labs < 32, lo, hi)
        keep = (word >> (labs & 31)) & 1
        x = inp_v[pl.ds(off, L)]
        inp_v[pl.ds(off, L)] = jnp.where(keep == 1, x, 0.0)
        return carry

    lax.fori_loop(0, CH // L, step, 0)
    pltpu.sync_copy(inp_v, out_hbm.at[pl.ds(base, CH)])


def kernel(input_tensor, label_tensor):
    key = jax.random.key(42)
    clear_mask = jax.random.bernoulli(key, 0.5, (NUM_LABELS,))
    clear_mask = clear_mask.at[0].set(False)
    keep_bits = jnp.where(clear_mask, 0, 1).astype(jnp.int32)
    shifts = jnp.arange(NUM_LABELS, dtype=jnp.int32) & 31
    packed = keep_bits << shifts
    lo = jnp.sum(packed[:32]).astype(jnp.int32)
    hi = jnp.sum(packed[32:]).astype(jnp.int32)
    words = jnp.concatenate(
        [jnp.full((L,), lo, jnp.int32), jnp.full((L,), hi, jnp.int32)]
    )
    out = _clear_body(
        input_tensor.reshape(N), label_tensor.reshape(N), words
    )
    return out.reshape(input_tensor.shape)


# speedup vs baseline: 196.6009x; 1.1240x over previous
import functools

import jax
import jax.numpy as jnp
from jax import lax
from jax.experimental import pallas as pl
from jax.experimental.pallas import tpu as pltpu
from jax.experimental.pallas import tpu_sc as plsc

NUM_LABELS = 64
N = 4 * 512 * 512
NC, NS, L = 2, 16, 16
NW = NC * NS
CH = N // NW


@functools.partial(
    pl.kernel,
    mesh=plsc.VectorSubcoreMesh(core_axis_name="c", subcore_axis_name="s"),
    out_type=jax.ShapeDtypeStruct((N,), jnp.float32),
    scratch_types=[
        pltpu.VMEM((CH,), jnp.float32),
        pltpu.VMEM((CH,), jnp.int32),
        pltpu.VMEM((2 * L,), jnp.int32),
    ],
)
def _clear_body(inp_hbm, lab_hbm, words_hbm, out_hbm, inp_v, lab_v, words_v):
    wid = lax.axis_index("s") * NC + lax.axis_index("c")
    base = wid * CH
    pltpu.sync_copy(words_hbm, words_v)
    pltpu.sync_copy(inp_hbm.at[pl.ds(base, CH)], inp_v)
    pltpu.sync_copy(lab_hbm.at[pl.ds(base, CH)], lab_v)
    lo = words_v[pl.ds(0, L)]
    hi = words_v[pl.ds(L, L)]

    @plsc.parallel_loop(0, CH, step=L, unroll=8)
    def _loop(off):
        labs = lab_v[pl.ds(off, L)]
        word = jnp.where(labs < 32, lo, hi)
        keep = (word >> (labs & 31)) & 1
        x = inp_v[pl.ds(off, L)]
        inp_v[pl.ds(off, L)] = jnp.where(keep == 1, x, 0.0)
    pltpu.sync_copy(inp_v, out_hbm.at[pl.ds(base, CH)])


def kernel(input_tensor, label_tensor):
    key = jax.random.key(42)
    clear_mask = jax.random.bernoulli(key, 0.5, (NUM_LABELS,))
    clear_mask = clear_mask.at[0].set(False)
    keep_bits = jnp.where(clear_mask, 0, 1).astype(jnp.int32)
    shifts = jnp.arange(NUM_LABELS, dtype=jnp.int32) & 31
    packed = keep_bits << shifts
    lo = jnp.sum(packed[:32]).astype(jnp.int32)
    hi = jnp.sum(packed[32:]).astype(jnp.int32)
    words = jnp.concatenate(
        [jnp.full((L,), lo, jnp.int32), jnp.full((L,), hi, jnp.int32)]
    )
    out = _clear_body(
        input_tensor.reshape(N), label_tensor.reshape(N), words
    )
    return out.reshape(input_tensor.shape)


# speedup vs baseline: 306.4356x; 1.5587x over previous
import functools

import jax
import jax.numpy as jnp
import numpy as np
from jax import lax
from jax.experimental import pallas as pl
from jax.experimental.pallas import tpu as pltpu
from jax.experimental.pallas import tpu_sc as plsc

NUM_LABELS = 64
B, H, W = 4, 512, 512
NC, NS, L = 2, 16, 16
NW = NC * NS
ROWS = B * H // NW
CH = ROWS * W

_LO = np.int32(np.uint32(0x728BBBAF))
_HI = np.int32(np.uint32(0x4C65DA36))


@functools.partial(
    pl.kernel,
    mesh=plsc.VectorSubcoreMesh(core_axis_name="c", subcore_axis_name="s"),
    out_type=jax.ShapeDtypeStruct((B, H, W), jnp.float32),
    scratch_types=[
        pltpu.VMEM((ROWS, W), jnp.float32),
        pltpu.VMEM((ROWS, W), jnp.int32),
    ],
)
def _clear_body(inp_hbm, lab_hbm, out_hbm, inp_v, lab_v):
    wid = lax.axis_index("s") * NC + lax.axis_index("c")
    img = wid // (H // ROWS)
    r0 = (wid % (H // ROWS)) * ROWS
    pltpu.sync_copy(inp_hbm.at[img, pl.ds(r0, ROWS)], inp_v)
    pltpu.sync_copy(lab_hbm.at[img, pl.ds(r0, ROWS)], lab_v)
    lo = jnp.full((L,), _LO, dtype=jnp.int32)
    hi = jnp.full((L,), _HI, dtype=jnp.int32)

    @plsc.parallel_loop(0, CH, step=L, unroll=8)
    def _loop(off):
        r = off // W
        c = off % W
        labs = lab_v[r, pl.ds(c, L)]
        word = jnp.where(labs < 32, lo, hi)
        keep = (word >> (labs & 31)) & 1
        x = inp_v[r, pl.ds(c, L)]
        inp_v[r, pl.ds(c, L)] = jnp.where(keep == 1, x, 0.0)

    pltpu.sync_copy(inp_v, out_hbm.at[img, pl.ds(r0, ROWS)])


def kernel(input_tensor, label_tensor):
    return _clear_body(input_tensor, label_tensor)


# speedup vs baseline: 319.0473x; 1.0412x over previous
import functools

import jax
import jax.numpy as jnp
import numpy as np
from jax import lax
from jax.experimental import pallas as pl
from jax.experimental.pallas import tpu as pltpu
from jax.experimental.pallas import tpu_sc as plsc

NUM_LABELS = 64
B, H, W = 4, 512, 512
NC, NS, L = 2, 16, 16
NW = NC * NS
ROWS = B * H // NW
CH = ROWS * W

_LO = np.int32(np.uint32(0x728BBBAF))
_HI = np.int32(np.uint32(0x4C65DA36))


@functools.partial(
    pl.kernel,
    mesh=plsc.VectorSubcoreMesh(core_axis_name="c", subcore_axis_name="s"),
    out_type=jax.ShapeDtypeStruct((B, H, W), jnp.float32),
    scratch_types=[
        pltpu.VMEM((2, ROWS // 2, W), jnp.float32),
        pltpu.VMEM((2, ROWS // 2, W), jnp.int32),
        pltpu.SemaphoreType.DMA,
        pltpu.SemaphoreType.DMA,
        pltpu.SemaphoreType.DMA,
    ],
)
def _clear_body(inp_hbm, lab_hbm, out_hbm, inp_v, lab_v, s0, s1, so):
    wid = lax.axis_index("s") * NC + lax.axis_index("c")
    img = wid // (H // ROWS)
    r0 = (wid % (H // ROWS)) * ROWS
    lo = jnp.full((L,), _LO, dtype=jnp.int32)
    hi = jnp.full((L,), _HI, dtype=jnp.int32)

    RC = ROWS // 2
    sems = (s0, s1)
    in_copies = []
    for k in range(2):
        in_copies.append(
            (
                pltpu.async_copy(
                    inp_hbm.at[img, pl.ds(r0 + k * RC, RC)], inp_v.at[k], sems[k]
                ),
                pltpu.async_copy(
                    lab_hbm.at[img, pl.ds(r0 + k * RC, RC)], lab_v.at[k], sems[k]
                ),
            )
        )
    out_copies = []
    for k in range(2):
        for c in in_copies[k]:
            c.wait()

        @plsc.parallel_loop(0, RC * W, step=L, unroll=8)
        def _loop(off):
            r = off // W
            c = off % W
            labs = lab_v[k, r, pl.ds(c, L)]
            word = jnp.where(labs < 32, lo, hi)
            keep = (word >> (labs & 31)) & 1
            x = inp_v[k, r, pl.ds(c, L)]
            inp_v[k, r, pl.ds(c, L)] = jnp.where(keep == 1, x, 0.0)

        out_copies.append(
            pltpu.async_copy(
                inp_v.at[k], out_hbm.at[img, pl.ds(r0 + k * RC, RC)], so
            )
        )
    for c in out_copies:
        c.wait()


def kernel(input_tensor, label_tensor):
    return _clear_body(input_tensor, label_tensor)


# speedup vs baseline: 319.7380x; 1.0022x over previous
import functools

import jax
import jax.numpy as jnp
import numpy as np
from jax import lax
from jax.experimental import pallas as pl
from jax.experimental.pallas import tpu as pltpu
from jax.experimental.pallas import tpu_sc as plsc

NUM_LABELS = 64
B, H, W = 4, 512, 512
NC, NS, L = 2, 16, 16
NW = NC * NS
ROWS = B * H // NW
CH = ROWS * W

_LO = np.int32(np.uint32(0x728BBBAF))
_HI = np.int32(np.uint32(0x4C65DA36))


@functools.partial(
    pl.kernel,
    mesh=plsc.VectorSubcoreMesh(core_axis_name="c", subcore_axis_name="s"),
    out_type=jax.ShapeDtypeStruct((B, H, W), jnp.float32),
    scratch_types=[
        pltpu.VMEM((2, ROWS // 2, W), jnp.float32),
        pltpu.VMEM((2, ROWS // 2, W), jnp.int32),
        pltpu.SemaphoreType.DMA,
        pltpu.SemaphoreType.DMA,
        pltpu.SemaphoreType.DMA,
    ],
)
def _clear_body(inp_hbm, lab_hbm, out_hbm, inp_v, lab_v, s0, s1, so):
    wid = lax.axis_index("s") * NC + lax.axis_index("c")
    img = wid // (H // ROWS)
    r0 = (wid % (H // ROWS)) * ROWS
    lo = jnp.full((L,), _LO, dtype=jnp.int32)
    hi = jnp.full((L,), _HI, dtype=jnp.int32)

    RC = ROWS // 2
    sems = (s0, s1)
    in_copies = []
    for k in range(2):
        in_copies.append(
            (
                pltpu.async_copy(
                    inp_hbm.at[img, pl.ds(r0 + k * RC, RC)], inp_v.at[k], sems[k]
                ),
                pltpu.async_copy(
                    lab_hbm.at[img, pl.ds(r0 + k * RC, RC)], lab_v.at[k], sems[k]
                ),
            )
        )
    out_copies = []
    for k in range(2):
        for c in in_copies[k]:
            c.wait()

        @plsc.parallel_loop(0, RC * W, step=L, unroll=4)
        def _loop(off):
            r = off // W
            c = off % W
            labs = lab_v[k, r, pl.ds(c, L)]
            word = jnp.where(labs < 32, lo, hi)
            keep = (word >> (labs & 31)) & 1
            x = inp_v[k, r, pl.ds(c, L)]
            inp_v[k, r, pl.ds(c, L)] = jnp.where(keep == 1, x, 0.0)

        out_copies.append(
            pltpu.async_copy(
                inp_v.at[k], out_hbm.at[img, pl.ds(r0 + k * RC, RC)], so
            )
        )
    for c in out_copies:
        c.wait()


def kernel(input_tensor, label_tensor):
    return _clear_body(input_tensor, label_tensor)
